# Initial kernel scaffold; baseline (speedup 1.0000x reference)
import jax
import jax.numpy as jnp
from jax.experimental import pallas as pl


def kernel(protein_x, protein_edge_index, protein_edge_attr, protein_batch, peptide_x, peptide_edge_index, peptide_edge_attr, peptide_batch, params):
    raise NotImplementedError("write your pallas kernel here")


# speedup vs baseline: 8.5790x; 8.5790x over previous
import functools

import jax
import jax.numpy as jnp
from jax import lax
from jax.experimental import pallas as pl
from jax.experimental.pallas import tpu as pltpu
from jax.experimental.pallas import tpu_sc as plsc

HEADS = 2
CH = 32
D = HEADS * CH
EDGE_DIM = 16
G = 64
G1 = G + 1
NC = 2
NS = 16
L = 16


def _node_pre_body(x_ref, w_ref, h_ref, a_ref, b_ref):
    t = jnp.dot(x_ref[...], w_ref[...], preferred_element_type=jnp.float32)
    h_ref[...] = t[:, :D]
    a_ref[...] = t[:, D:D + 2]
    b_ref[...] = t[:, D + 2:D + 4]


def _node_pre(x, w_ext, bn):
    n, f = x.shape
    return pl.pallas_call(
        _node_pre_body,
        grid=(n // bn,),
        in_specs=[pl.BlockSpec((bn, f), lambda i: (i, 0)),
                  pl.BlockSpec((f, D + 4), lambda i: (0, 0))],
        out_specs=[pl.BlockSpec((bn, D), lambda i: (i, 0)),
                   pl.BlockSpec((bn, 2), lambda i: (i, 0)),
                   pl.BlockSpec((bn, 2), lambda i: (i, 0))],
        out_shape=[jax.ShapeDtypeStruct((n, D), jnp.float32),
                   jax.ShapeDtypeStruct((n, 2), jnp.float32),
                   jax.ShapeDtypeStruct((n, 2), jnp.float32)],
    )(x, w_ext)


def _edge_pre_body(ea_ref, w_ref, out_ref):
    out_ref[...] = jnp.dot(ea_ref[...], w_ref[...],
                           preferred_element_type=jnp.float32)


def _edge_pre(ea, ae_w, be):
    e = ea.shape[0]
    return pl.pallas_call(
        _edge_pre_body,
        grid=(e // be,),
        in_specs=[pl.BlockSpec((be, EDGE_DIM), lambda i: (i, 0)),
                  pl.BlockSpec((EDGE_DIM, 2), lambda i: (0, 0))],
        out_specs=pl.BlockSpec((be, 2), lambda i: (i, 0)),
        out_shape=jax.ShapeDtypeStruct((e, 2), jnp.float32),
    )(ea, ae_w)


def _post_body(acc_ref, den_ref, bias_ref, g_ref, b_ref, out_ref):
    d = den_ref[...]
    o0 = acc_ref[:, :CH] / (d[:, 0:1] + 1e-16)
    o1 = acc_ref[:, CH:] / (d[:, 1:2] + 1e-16)
    o = jnp.concatenate([o0, o1], axis=1) + bias_ref[...]
    m = jnp.mean(o, axis=-1, keepdims=True)
    v = jnp.mean((o - m) ** 2, axis=-1, keepdims=True)
    o = (o - m) / jnp.sqrt(v + 1e-5) * g_ref[...] + b_ref[...]
    out_ref[...] = jnp.maximum(o, 0.0)


def _post(acc, den, bias, ln_g, ln_b, bn):
    n = acc.shape[0]
    vec = pl.BlockSpec((1, D), lambda i: (0, 0))
    return pl.pallas_call(
        _post_body,
        grid=(n // bn,),
        in_specs=[pl.BlockSpec((bn, D), lambda i: (i, 0)),
                  pl.BlockSpec((bn, 2), lambda i: (i, 0)),
                  vec, vec, vec],
        out_specs=pl.BlockSpec((bn, D), lambda i: (i, 0)),
        out_shape=jax.ShapeDtypeStruct((n, D), jnp.float32),
    )(acc, den, bias.reshape(1, D), ln_g.reshape(1, D), ln_b.reshape(1, D))


def _head_body(ppp_ref, ppc_ref, qpp_ref, qpc_ref, wv_ref, bv_ref,
               wo_ref, bo_ref, g_ref, b_ref, out_ref):
    pc = jnp.maximum(jnp.sum(ppc_ref[...], axis=0)[:, 0:1], 1.0)
    qc = jnp.maximum(jnp.sum(qpc_ref[...], axis=0)[:, 0:1], 1.0)
    pg = jnp.sum(ppp_ref[...], axis=0) / pc
    qg = jnp.sum(qpp_ref[...], axis=0) / qc
    ao = jnp.dot(qg, wv_ref[...], preferred_element_type=jnp.float32) + bv_ref[...]
    ao = jnp.dot(ao, wo_ref[...], preferred_element_type=jnp.float32) + bo_ref[...]
    f = ao + pg
    m = jnp.mean(f, axis=-1, keepdims=True)
    v = jnp.mean((f - m) ** 2, axis=-1, keepdims=True)
    out_ref[...] = (f - m) / jnp.sqrt(v + 1e-5) * g_ref[...] + b_ref[...]


def _head(ppp, ppc, qpp, qpc, mha, fn_g, fn_b):
    return pl.pallas_call(
        _head_body,
        out_shape=jax.ShapeDtypeStruct((G, D), jnp.float32),
    )(ppp, ppc, qpp, qpc, mha['Wv'],
      mha['bv'].reshape(1, D), mha['Wo'], mha['bo'].reshape(1, D),
      fn_g.reshape(1, D), fn_b.reshape(1, D))


def _edge_sc_call(n_pad, e_pad, chunk, nranges):
    HC = CH // 2
    HR = n_pad // nranges
    epw = e_pad // NS
    nchunks = epw // chunk
    grp = chunk // L
    nrw = HR // NS
    zc = next(z for z in range(min(chunk, nrw), 0, -1)
              if nrw % z == 0 and z % 8 == 0)
    nzc = nrw // zc

    mesh = plsc.VectorSubcoreMesh(core_axis_name="c", subcore_axis_name="s",
                                  num_cores=NC, num_subcores=NS)

    @functools.partial(
        pl.kernel,
        out_type=(pltpu.HBM((NC, 2, n_pad, HC), jnp.float32),
                  pltpu.HBM((NC, n_pad), jnp.float32)),
        mesh=mesh,
        scratch_types=[
            pltpu.VMEM((n_pad,), jnp.float32),
            pltpu.VMEM((n_pad,), jnp.float32),
            pltpu.VMEM((chunk,), jnp.int32),
            pltpu.VMEM((chunk,), jnp.int32),
            pltpu.VMEM((chunk,), jnp.int32),
            pltpu.VMEM((chunk,), jnp.float32),
            pltpu.VMEM((chunk,), jnp.float32),
            pltpu.VMEM((chunk, HC), jnp.float32),
            pltpu.VMEM((chunk, HC), jnp.float32),
            pltpu.VMEM((chunk,), jnp.float32),
            pltpu.VMEM_SHARED((HR + 8, HC), jnp.float32),
            pltpu.VMEM_SHARED((HR + 8,), jnp.float32),
            pltpu.SemaphoreType.DMA,
        ],
        compiler_params=pltpu.CompilerParams(needs_layout_passes=False,
                                             use_tc_tiling_on_sc=False),
    )
    def k(src_h, dst_h, ae_h, as_h, ad_h, hh_h, acc_o, den_o,
          as_v, ad_v, srcb, dstb, sdb, aeb, exb, rows, zb, zd, out_s, den_s,
          sem):
        c = lax.axis_index("c")
        t = lax.axis_index("s")

        pltpu.sync_copy(as_h.at[c], as_v)
        pltpu.sync_copy(ad_h.at[c], ad_v)

        def zrow(i, _):
            zb[i, 0:L] = jnp.zeros((L,), jnp.float32)
            return 0
        lax.fori_loop(0, chunk, zrow, 0)

        def zex(i, _):
            zd[pl.ds(i * L, L)] = jnp.zeros((L,), jnp.float32)
            return 0
        lax.fori_loop(0, grp, zex, 0)

        for half in range(2):
            for rng in range(nranges):
                if half == 0:
                    def zden(j, _):
                        pltpu.sync_copy(
                            zd.at[pl.ds(0, zc)],
                            den_s.at[pl.ds(t * nrw + j * zc, zc)])
                        return 0
                    lax.fori_loop(0, nzc, zden, 0)
                def zfill(j, _):
                    pltpu.sync_copy(
                        zb.at[pl.ds(0, zc)],
                        out_s.at[pl.ds(t * nrw + j * zc, zc)])
                    return 0
                lax.fori_loop(0, nzc, zfill, 0)

                plsc.subcore_barrier()

                lo = rng * HR

                def body(i, _):
                    base = t * epw + i * chunk
                    pltpu.sync_copy(src_h.at[pl.ds(base, chunk)], srcb)
                    pltpu.sync_copy(dst_h.at[pl.ds(base, chunk)], dstb)
                    pltpu.sync_copy(ae_h.at[c, pl.ds(base, chunk)], aeb)
                    pltpu.async_copy(hh_h.at[c, half].at[srcb], rows,
                                     sem).wait()

                    def vec(g, _):
                        sv = srcb[pl.ds(g * L, L)]
                        dv = dstb[pl.ds(g * L, L)]
                        pre = (plsc.load_gather(as_v, [sv]) +
                               plsc.load_gather(ad_v, [dv]) +
                               aeb[pl.ds(g * L, L)])
                        alpha = jnp.maximum(pre, 0.2 * pre)
                        exb[pl.ds(g * L, L)] = jnp.exp(alpha)
                        sd = dv - lo
                        ok = (sd >= 0) & (sd < HR)
                        sdb[pl.ds(g * L, L)] = jnp.where(
                            ok, sd, jnp.int32(HR))
                        return 0
                    lax.fori_loop(0, grp, vec, 0)

                    def scale(e2, _):
                        w16 = plsc.load_gather(
                            exb, [jnp.zeros((L,), jnp.int32) + e2])
                        rows[e2, 0:L] = rows[e2, 0:L] * w16
                        return 0
                    lax.fori_loop(0, chunk, scale, 0)

                    if half == 0:
                        pltpu.sync_copy(exb, den_s.at[sdb], add=True)
                    pltpu.sync_copy(rows, out_s.at[sdb], add=True)
                    return 0
                lax.fori_loop(0, nchunks, body, 0)

                plsc.subcore_barrier()

                row0 = t * nrw
                pltpu.sync_copy(out_s.at[pl.ds(row0, nrw)],
                                acc_o.at[c, half, pl.ds(lo + row0, nrw)])
                if half == 0:
                    pltpu.sync_copy(den_s.at[pl.ds(row0, nrw)],
                                    den_o.at[c, pl.ds(lo + row0, nrw)])
                if not (half == 1 and rng == nranges - 1):
                    plsc.subcore_barrier()

    return k


def _pool_sc_call(n_pad, chunk):
    npw = n_pad // (NC * NS)
    nchunks = npw // chunk
    KD = D // L

    mesh = plsc.VectorSubcoreMesh(core_axis_name="c", subcore_axis_name="s",
                                  num_cores=NC, num_subcores=NS)

    @functools.partial(
        pl.kernel,
        out_type=(pltpu.HBM((NC * NS, G1, D), jnp.float32),
                  pltpu.HBM((NC * NS, G1, L), jnp.float32)),
        mesh=mesh,
        scratch_types=[
            pltpu.VMEM((chunk, D), jnp.float32),
            pltpu.VMEM((chunk,), jnp.int32),
            pltpu.VMEM((G1, D), jnp.float32),
            pltpu.VMEM((G1, L), jnp.float32),
        ],
        compiler_params=pltpu.CompilerParams(needs_layout_passes=False),
    )
    def k(x_h, b_h, pp_o, pc_o, rowsb, batchb, pool_v, cnt_v):
        c = lax.axis_index("c")
        t = lax.axis_index("s")
        wid = c * NS + t

        def fill(i, _):
            r = i // KD
            kk = i % KD
            pool_v[r, pl.ds(kk * L, L)] = jnp.zeros((L,), jnp.float32)
            return 0
        lax.fori_loop(0, G1 * KD, fill, 0)

        def fillc(i, _):
            cnt_v[i, 0:L] = jnp.zeros((L,), jnp.float32)
            return 0
        lax.fori_loop(0, G1, fillc, 0)

        def body(i, _):
            base = (wid) * npw + i * chunk
            pltpu.sync_copy(x_h.at[pl.ds(base, chunk)], rowsb)
            pltpu.sync_copy(b_h.at[pl.ds(base, chunk)], batchb)

            def node(g, _):
                b16 = batchb[pl.ds(g * L, L)]
                for lane in range(L):
                    b = b16[lane]
                    j = g * L + lane
                    for kk in range(KD):
                        pool_v[b, pl.ds(kk * L, L)] = (
                            pool_v[b, pl.ds(kk * L, L)] +
                            rowsb[j, pl.ds(kk * L, L)])
                    cnt_v[b, 0:L] = cnt_v[b, 0:L] + 1.0
                return 0
            lax.fori_loop(0, chunk // L, node, 0)
            return 0
        lax.fori_loop(0, nchunks, body, 0)

        pltpu.sync_copy(pool_v, pp_o.at[wid])
        pltpu.sync_copy(cnt_v, pc_o.at[wid])

    return k


def _fold_gat(p):
    f = p['W'].shape[0]
    w3 = p['W'].reshape(f, HEADS, CH)
    a_s = jnp.sum(w3 * p['att_src'][None], axis=-1)
    a_d = jnp.sum(w3 * p['att_dst'][None], axis=-1)
    a_e = jnp.sum(p['W_e'].reshape(EDGE_DIM, HEADS, CH) *
                  p['att_edge'][None], axis=-1)
    return jnp.concatenate([p['W'], a_s, a_d], axis=1), a_e


def _encode(x, ei, ea, batch, layers, cfg):
    n, e = cfg['n'], cfg['e']
    n_pad, e_pad, chunk = cfg['n_pad'], cfg['e_pad'], cfg['chunk']
    src = ei[0]
    dst = ei[1]
    if e_pad > e:
        src = jnp.concatenate([src, jnp.zeros((e_pad - e,), jnp.int32)])
        dst = jnp.concatenate([dst, jnp.full((e_pad - e,), n, jnp.int32)])
    edge_call = _edge_sc_call(n_pad, e_pad, chunk, cfg['nranges'])

    for p in layers:
        w_ext, ae_w = _fold_gat(p)
        h, a2, b2 = _node_pre(x, w_ext, cfg['bn'])
        ae2 = _edge_pre(ea, ae_w, cfg['be'])
        as_t = jnp.pad(a2, ((0, n_pad - n), (0, 0))).T
        ad_t = jnp.pad(b2, ((0, n_pad - n), (0, 0))).T
        ae_t = jnp.pad(ae2, ((0, e_pad - e), (0, 0))).T
        hh = jnp.transpose(
            jnp.pad(h, ((0, n_pad - n), (0, 0))).reshape(n_pad, HEADS, 2,
                                                         CH // 2),
            (1, 2, 0, 3))
        acc, den = edge_call(src, dst, ae_t, as_t, ad_t, hh)
        acc_t = jnp.transpose(acc[:, :, :n, :], (2, 0, 1, 3)).reshape(n, D)
        den_t = den[:, :n].T
        x = _post(acc_t, den_t, p['bias'], p['ln_g'], p['ln_b'], cfg['bn'])

    npp = cfg['n_pool_pad']
    xf = jnp.pad(x, ((0, npp - n), (0, 0)))
    bp = jnp.concatenate([batch, jnp.full((npp - n,), G, jnp.int32)])
    pp, pc = _pool_sc_call(npp, cfg['pool_chunk'])(xf, bp)
    return pp, pc


_PROT_CFG = dict(n=50000, e=800000, n_pad=51200, e_pad=800768, chunk=128, nranges=4,
                 n_pool_pad=50176, pool_chunk=32, bn=2000, be=8000)
_PEP_CFG = dict(n=2048, e=16384, n_pad=2048, e_pad=16384, chunk=128, nranges=2,
                n_pool_pad=2048, pool_chunk=64, bn=2048, be=16384)


def kernel(protein_x, protein_edge_index, protein_edge_attr, protein_batch,
           peptide_x, peptide_edge_index, peptide_edge_attr, peptide_batch,
           params):
    ppp, ppc = _encode(protein_x, protein_edge_index, protein_edge_attr,
                       protein_batch, params['prot'], _PROT_CFG)
    qpp, qpc = _encode(peptide_x, peptide_edge_index, peptide_edge_attr,
                       peptide_batch, params['pep'], _PEP_CFG)
    return _head(ppp[:, :G, :], ppc[:, :G, :], qpp[:, :G, :], qpc[:, :G, :],
                 params['mha'], params['fn_g'], params['fn_b'])
